# baseline (speedup 1.0000x reference)
import functools

import jax
import jax.numpy as jnp
from jax import lax
from jax.experimental import pallas as pl
from jax.experimental.pallas import tpu as pltpu
from jax.experimental.pallas import tpu_sc as plsc

B = 16384
EMB = 16
M = 128
GRP = 128 // EMB

_NC = 2
_NS = 16
_NW = _NC * _NS
_BPW = B // _NW
_CH = 128

_sc_mesh = plsc.VectorSubcoreMesh(core_axis_name="c", subcore_axis_name="s")


@functools.partial(
    pl.kernel,
    out_type=(
        jax.ShapeDtypeStruct((B, 128), jnp.float32),
        jax.ShapeDtypeStruct((B, 128), jnp.float32),
    ),
    mesh=_sc_mesh,
    scratch_types=[
        pltpu.VMEM((_BPW,), jnp.int32),
        pltpu.VMEM((_BPW,), jnp.int32),
        pltpu.VMEM((_BPW,), jnp.int32),
        pltpu.VMEM((_BPW,), jnp.int32),
        pltpu.VMEM((_CH, 128), jnp.float32),
        pltpu.VMEM((_CH, 128), jnp.float32),
        pltpu.SemaphoreType.DMA,
        pltpu.SemaphoreType.DMA,
    ],
)
def _sc_gather(user_hbm, movie_hbm, uid_hbm, mid_hbm, ueb_hbm, meb_hbm,
               uidx_v, midx_v, urow_v, mrow_v, buf0, buf1, sem0, sem1):
    wid = lax.axis_index("s") * _NC + lax.axis_index("c")
    base = wid * _BPW
    pltpu.sync_copy(uid_hbm.at[pl.ds(base, _BPW)], uidx_v)
    pltpu.sync_copy(mid_hbm.at[pl.ds(base, _BPW)], midx_v)
    for blk in range(_BPW // 16):
        sl = pl.ds(blk * 16, 16)
        urow_v[sl] = lax.shift_right_logical(uidx_v[sl], 3)
        mrow_v[sl] = lax.shift_right_logical(midx_v[sl], 3)
    bufs = (buf0, buf1)
    sems = (sem0, sem1)
    nch = _BPW // _CH
    chunks = [(user_hbm, urow_v, ueb_hbm, c) for c in range(nch)]
    chunks += [(movie_hbm, mrow_v, meb_hbm, c) for c in range(nch)]
    cps = [None] * len(chunks)
    for i, (tbl, row_v, out_hbm, c) in enumerate(chunks):
        cps[i] = pltpu.async_copy(
            tbl.at[row_v.at[pl.ds(c * _CH, _CH)]], bufs[i % 2], sems[i % 2])
        if i > 0:
            _, _, pout, pc = chunks[i - 1]
            cps[i - 1].wait()
            pltpu.sync_copy(bufs[(i - 1) % 2],
                            pout.at[pl.ds(base + pc * _CH, _CH)])
    cps[-1].wait()
    _, _, pout, pc = chunks[-1]
    pltpu.sync_copy(bufs[(len(chunks) - 1) % 2],
                    pout.at[pl.ds(base + pc * _CH, _CH)])


_BLK = 2048


def _mlp_body(gu_ref, gm_ref, uid_ref, mid_ref, w1u_ref, w1m_ref, b1_ref,
              w2_ref, b2_ref, w3_ref, b3_ref, o_ref):
    grp = jax.lax.broadcasted_iota(jnp.int32, (_BLK, 128), 1) // EMB
    umask = (grp == uid_ref[...] % GRP).astype(jnp.float32)
    mmask = (grp == mid_ref[...] % GRP).astype(jnp.float32)
    h1 = (jnp.dot(gu_ref[...] * umask, w1u_ref[...],
                  preferred_element_type=jnp.float32)
          + jnp.dot(gm_ref[...] * mmask, w1m_ref[...],
                    preferred_element_type=jnp.float32)
          + b1_ref[...])
    h1 = jnp.maximum(h1, 0.0)
    h2 = jnp.maximum(
        jnp.dot(h1, w2_ref[...], preferred_element_type=jnp.float32)
        + b2_ref[...], 0.0)
    o_ref[...] = (jnp.dot(h2, w3_ref[...], preferred_element_type=jnp.float32)
                  + b3_ref[...])


_mlp = pl.pallas_call(
    _mlp_body,
    grid=(B // _BLK,),
    in_specs=[
        pl.BlockSpec((_BLK, 128), lambda i: (i, 0)),
        pl.BlockSpec((_BLK, 128), lambda i: (i, 0)),
        pl.BlockSpec((_BLK, 1), lambda i: (i, 0)),
        pl.BlockSpec((_BLK, 1), lambda i: (i, 0)),
        pl.BlockSpec((128, M), lambda i: (0, 0)),
        pl.BlockSpec((128, M), lambda i: (0, 0)),
        pl.BlockSpec((1, M), lambda i: (0, 0)),
        pl.BlockSpec((M, M // 2), lambda i: (0, 0)),
        pl.BlockSpec((1, M // 2), lambda i: (0, 0)),
        pl.BlockSpec((M // 2, 1), lambda i: (0, 0)),
        pl.BlockSpec((1, 1), lambda i: (0, 0)),
    ],
    out_specs=pl.BlockSpec((_BLK, 1), lambda i: (i, 0)),
    out_shape=jax.ShapeDtypeStruct((B, 1), jnp.float32),
)


def kernel(userId, movieId, user_table, movie_table, W1, b1, W2, b2, W3, b3):
    ut = user_table.reshape(-1, GRP, EMB).reshape(-1, 128)
    mt = movie_table.reshape(-1, GRP, EMB).reshape(-1, 128)
    gu, gm = _sc_gather(ut, mt, userId, movieId)
    w1t = W1.T
    w1u = jnp.tile(w1t[:EMB], (GRP, 1))
    w1m = jnp.tile(w1t[EMB:], (GRP, 1))
    return _mlp(gu, gm, userId.reshape(B, 1), movieId.reshape(B, 1),
                w1u, w1m, b1.reshape(1, M), W2.T, b2.reshape(1, M // 2),
                W3.T, b3.reshape(1, 1))


# speedup vs baseline: 3.0786x; 3.0786x over previous
import functools

import jax
import jax.numpy as jnp
from jax import lax
from jax.experimental import pallas as pl
from jax.experimental.pallas import tpu as pltpu
from jax.experimental.pallas import tpu_sc as plsc

B = 16384
EMB = 16
M = 128

_NC = 2
_NS = 16
_NW = _NC * _NS
_BPW = B // _NW
_NBUF = 8

_sc_mesh = plsc.VectorSubcoreMesh(core_axis_name="c", subcore_axis_name="s")


@functools.partial(
    pl.kernel,
    out_type=jax.ShapeDtypeStruct((2 * EMB, B), jnp.float32),
    mesh=_sc_mesh,
    scratch_types=[
        pltpu.VMEM((_BPW,), jnp.int32),
        pltpu.VMEM((_BPW,), jnp.int32),
        pltpu.VMEM((_BPW // 16, 16), jnp.int32),
        pltpu.VMEM((_BPW // 16, 16), jnp.int32),
        pltpu.VMEM((2 * EMB, _BPW), jnp.float32),
        [pltpu.VMEM((EMB, 128), jnp.float32) for _ in range(_NBUF)],
        [pltpu.VMEM((EMB, 128), jnp.float32) for _ in range(_NBUF)],
        [pltpu.SemaphoreType.DMA for _ in range(_NBUF)],
        [pltpu.SemaphoreType.DMA for _ in range(_NBUF)],
    ],
    compiler_params=pltpu.CompilerParams(needs_layout_passes=False),
)
def _sc_gather_t(ut_hbm, mt_hbm, uid_hbm, mid_hbm, x_hbm,
                 uid_v, mid_v, uid2, mid2, xbuf, ubufs, mbufs, usems, msems):
    wid = lax.axis_index("s") * _NC + lax.axis_index("c")
    base = pl.multiple_of(wid * _BPW, 128)
    pltpu.sync_copy(uid_hbm.at[pl.ds(base, _BPW)], uid_v)
    pltpu.sync_copy(mid_hbm.at[pl.ds(base, _BPW)], mid_v)
    for c in range(_BPW // 16):
        uid2[c, :] = uid_v[pl.ds(c * 16, 16)]
        mid2[c, :] = mid_v[pl.ds(c * 16, 16)]

    lanes = lax.iota(jnp.int32, 16)

    def sid(ref2, i):
        chunk = ref2[lax.shift_right_logical(i, 4), :]
        sel = jnp.where(lanes == lax.bitwise_and(i, 15), chunk, -1)
        return jnp.max(sel)

    def fire(i, b):
        u = sid(uid2, i)
        m = sid(mid2, i)
        uoff = pl.multiple_of(lax.shift_right_logical(u, 7) * 128, 128)
        moff = pl.multiple_of(lax.shift_right_logical(m, 7) * 128, 128)
        pltpu.async_copy(ut_hbm.at[:, pl.ds(uoff, 128)], ubufs[b], usems[b])
        pltpu.async_copy(mt_hbm.at[:, pl.ds(moff, 128)], mbufs[b], msems[b])
        return u, m

    ids = [fire(b, b) for b in range(_NBUF)]

    rows = lax.iota(jnp.int32, 16)

    def process(u, m, i, b):
        pltpu.make_async_copy(ut_hbm.at[:, pl.ds(0, 128)],
                              ubufs[b], usems[b]).wait()
        pltpu.make_async_copy(mt_hbm.at[:, pl.ds(0, 128)],
                              mbufs[b], msems[b]).wait()
        ulane = jnp.full((16,), lax.rem(u, 128), jnp.int32)
        mlane = jnp.full((16,), lax.rem(m, 128), jnp.int32)
        col = jnp.full((16,), i, jnp.int32)
        uvals = plsc.load_gather(ubufs[b], [rows, ulane])
        mvals = plsc.load_gather(mbufs[b], [rows, mlane])
        plsc.store_scatter(xbuf, [rows, col], uvals)
        plsc.store_scatter(xbuf, [rows + EMB, col], mvals)

    def body(g, carry):
        carry_ids = carry
        new_ids = []
        for b in range(_NBUF):
            i = g * _NBUF + b
            u, m = carry_ids[b]
            process(u, m, i, b)
            nxt = jnp.minimum(i + _NBUF, _BPW - 1)
            new_ids.append(fire(nxt, b))
        return tuple(new_ids)

    lax.fori_loop(0, _BPW // _NBUF, body, tuple(ids))
    for b in range(_NBUF):
        pltpu.make_async_copy(ut_hbm.at[:, pl.ds(0, 128)],
                              ubufs[b], usems[b]).wait()
        pltpu.make_async_copy(mt_hbm.at[:, pl.ds(0, 128)],
                              mbufs[b], msems[b]).wait()
    pltpu.sync_copy(xbuf, x_hbm.at[:, pl.ds(base, _BPW)])


_BLK = 2048


def _mlp_body(x_ref, w1_ref, b1_ref, w2_ref, b2_ref, w3_ref, b3_ref, o_ref):
    h1 = jnp.maximum(
        jnp.dot(w1_ref[...], x_ref[...], preferred_element_type=jnp.float32)
        + b1_ref[...], 0.0)
    h2 = jnp.maximum(
        jnp.dot(w2_ref[...], h1, preferred_element_type=jnp.float32)
        + b2_ref[...], 0.0)
    o_ref[...] = (jnp.dot(w3_ref[...], h2, preferred_element_type=jnp.float32)
                  + b3_ref[...])


_mlp = pl.pallas_call(
    _mlp_body,
    grid=(B // _BLK,),
    in_specs=[
        pl.BlockSpec((2 * EMB, _BLK), lambda i: (0, i)),
        pl.BlockSpec((M, 2 * EMB), lambda i: (0, 0)),
        pl.BlockSpec((M, 1), lambda i: (0, 0)),
        pl.BlockSpec((M // 2, M), lambda i: (0, 0)),
        pl.BlockSpec((M // 2, 1), lambda i: (0, 0)),
        pl.BlockSpec((1, M // 2), lambda i: (0, 0)),
        pl.BlockSpec((1, 1), lambda i: (0, 0)),
    ],
    out_specs=pl.BlockSpec((1, _BLK), lambda i: (0, i)),
    out_shape=jax.ShapeDtypeStruct((1, B), jnp.float32),
)


def kernel(userId, movieId, user_table, movie_table, W1, b1, W2, b2, W3, b3):
    xt = _sc_gather_t(user_table.T, movie_table.T, userId, movieId)
    out_t = _mlp(xt, W1, b1.reshape(M, 1), W2, b2.reshape(M // 2, 1),
                 W3, b3.reshape(1, 1))
    return out_t.T
